# baseline (speedup 1.0000x reference)
import functools
import numpy as np
import jax
import jax.numpy as jnp
from jax import lax
from jax.experimental import pallas as pl
from jax.experimental.pallas import tpu as pltpu, tpu_sc as plsc

B = 1024
D = 32
NC, NS = 2, 16
NW = NC * NS
RPW = B // NW
CSZ = 128
NBUF = 3

FEATS = [
    (0, 5, 0), (1, 20, 1), (2, 20, 2), (3, 1, 3), (4, 20, 4), (5, 4, 5),
    (6, 26, 6), (7, 10, 7), (8, 10, 8), (9, 10, 9), (10, 10, 10),
    (11, 10, 11), (12, 10, 12), (13, 10, 13), (14, 10, 14),
    (15, 1, 15), (16, 1, 16), (17, 1, 17),
    (18, 50, 15), (19, 50, 15), (20, 50, 15),
    (21, 50, 17), (22, 50, 17), (23, 50, 17),
    (24, 20, 15), (25, 20, 15), (26, 20, 15),
    (27, 20, 17), (28, 20, 17), (29, 20, 17),
]
NCOLS = sum(n for _, n, _ in FEATS)

_off = 0
COL_OFF = []
for _, n, _ in FEATS:
    COL_OFF.append(_off)
    _off += n

TBL_FEATS = [[] for _ in range(18)]
for fi, (pos, n, t) in enumerate(FEATS):
    TBL_FEATS[t].append((pos, n, COL_OFF[fi]))
NT = [sum(n for _, n, _ in fs) for fs in TBL_FEATS]
CNT = [-(-RPW * nt // CSZ) for nt in NT]

GROUP_B = [15]
GROUP_A = [t for t in range(18) if t not in GROUP_B]

BIG_TABLES = [2, 6, 15, 16]

_BC = 512


@functools.cache
def _tconv_kernel(V):
    mb = 8
    step = mb * _BC
    nfull, rem = divmod(V, step)
    grid = nfull + (1 if rem else 0)

    def tbody(x_ref, tail_ref, o_ref):
        x8 = x_ref[...]
        if rem:
            x8 = jnp.where(pl.program_id(0) == grid - 1, tail_ref[...], x8)
        for k in range(mb):
            y = jnp.transpose(x8[:, k * _BC:(k + 1) * _BC])
            y4 = y.reshape(_BC // 4, 4, D)
            o_ref[k * (_BC // 4):(k + 1) * (_BC // 4), :] = (
                jnp.concatenate([y4[:, a, :] for a in range(4)], axis=1))

    return pl.pallas_call(
        tbody,
        grid=(grid,),
        in_specs=[pl.BlockSpec((32, step), lambda c: (0, c)),
                  pl.BlockSpec((32, step), lambda c: (0, 0))],
        out_specs=pl.BlockSpec((step // 4, 128), lambda c: (c, 0)),
        out_shape=jax.ShapeDtypeStruct((V * D // 128, 128), jnp.float32),
    )


def _to_rowmajor(T):
    V = T.shape[0]
    Tt = jnp.transpose(T)
    step = 8 * _BC
    rem = V % step
    if rem:
        tail = jnp.pad(Tt[:, V - rem:], ((0, 0), (0, step - rem)))
    else:
        tail = jnp.zeros((D, step), jnp.float32)
    q = _tconv_kernel(V)(Tt, tail)
    return jnp.reshape(q, (V, D))


def _pad_worker_chunks(a):
    nt = a.shape[1]
    a = a.reshape(NW, RPW * nt)
    pad = -(-RPW * nt // CSZ) * CSZ - RPW * nt
    if pad:
        mod = jnp if isinstance(a, jax.Array) else np
        a = mod.pad(a, ((0, 0), (0, pad)), mode="edge")
    return a.reshape(NW, -1, CSZ)


def _dst_part(t):
    cols = np.concatenate([off + np.arange(n) for _, n, off in TBL_FEATS[t]])
    dst = (np.arange(B) * NCOLS)[:, None] + cols[None, :]
    return _pad_worker_chunks(dst.astype(np.int32))


_DST = {g: np.concatenate([_dst_part(t) for t in grp], axis=1)
        for g, grp in (("A", GROUP_A), ("B", GROUP_B))}


def _pipeline(tables, out_hbm, src_v, dst_v, bufs, gsems, ssems, tbl_of):
    n = len(tbl_of)

    def gather(i):
        s = i % NBUF
        return pltpu.async_copy(
            tables[tbl_of[i]].at[src_v.at[i]], bufs[s], gsems[s])

    def scatter(i):
        s = i % NBUF
        return pltpu.async_copy(bufs[s], out_hbm.at[dst_v.at[i]], ssems[s])

    g = [None] * n
    sc = [None] * n
    for i in range(min(NBUF - 1, n)):
        g[i] = gather(i)
    for i in range(n):
        if i >= 1:
            sc[i - 1].wait()
        if i + NBUF - 1 < n:
            g[i + NBUF - 1] = gather(i + NBUF - 1)
        g[i].wait()
        sc[i] = scatter(i)
    sc[n - 1].wait()


def _make_kernel(group, produce_out):
    grp = GROUP_A if group == "A" else GROUP_B
    ntab = len(grp)
    tbl_of = []
    for k, t in enumerate(grp):
        tbl_of += [k] * CNT[t]
    ch = len(tbl_of)
    mesh = plsc.VectorSubcoreMesh(core_axis_name="c", subcore_axis_name="s")

    @functools.partial(
        pl.kernel,
        out_type=(jax.ShapeDtypeStruct((B * NCOLS, D), jnp.float32)
                  if produce_out else ()),
        mesh=mesh,
        compiler_params=pltpu.CompilerParams(use_tc_tiling_on_sc=False),
        scratch_types=[
            pltpu.VMEM((ch, CSZ), jnp.int32),
            pltpu.VMEM((ch, CSZ), jnp.int32),
        ] + [pltpu.VMEM((CSZ, D), jnp.float32)] * NBUF
          + [pltpu.SemaphoreType.DMA] * (2 * NBUF),
    )
    def body(src_hbm, dst_hbm, *rest):
        tables = rest[:ntab]
        out_hbm = rest[ntab]
        src_v, dst_v = rest[ntab + 1:ntab + 3]
        bufs = rest[ntab + 3:ntab + 3 + NBUF]
        gsems = rest[ntab + 3 + NBUF:ntab + 3 + 2 * NBUF]
        ssems = rest[ntab + 3 + 2 * NBUF:ntab + 3 + 3 * NBUF]
        w = lax.axis_index("s") * NC + lax.axis_index("c")
        pltpu.sync_copy(src_hbm.at[w], src_v)
        pltpu.sync_copy(dst_hbm.at[w], dst_v)
        _pipeline(tables, out_hbm, src_v, dst_v, bufs, gsems, ssems, tbl_of)

    return body


@functools.cache
def _kernels():
    return _make_kernel("A", True), _make_kernel("B", False)


def kernel(context_features, realtime_back_category, realtime_goods, realtime_pair_click, realtime_passtime, realtime_user_group, goods_sparse, bucket_user_box_obj, bucket_goods_box_obj, bucket_goods_gross_obj, pair_feature, bucket_pair_box_obj, bucket_user_cspu_obj, bucket_ozid_cspu_obj, bucket_user_behavior_obj, cspu_idx, supplier_idx, lv2_idx, long_click, long_cart, long_buy, long_buy_level2, long_cart_level2, long_click_level2, short_click, short_cart, short_buy, short_click_level2, short_cart_level2, short_buy_level2, T_context, T_back_cat, T_goods_rt, T_pair_click, T_passtime, T_user_group, T_goods, T_bucket_user, T_bucket_goods, T_bucket_goods_gross, T_pair, T_bucket_pair, T_bucket_user_cspu, T_bucket_ozid_cspu, T_bucket_user_behavior, T_cspu, T_supplier, T_level2):
    idxs = [context_features, realtime_back_category, realtime_goods, realtime_pair_click, realtime_passtime, realtime_user_group, goods_sparse, bucket_user_box_obj, bucket_goods_box_obj, bucket_goods_gross_obj, pair_feature, bucket_pair_box_obj, bucket_user_cspu_obj, bucket_ozid_cspu_obj, bucket_user_behavior_obj, cspu_idx, supplier_idx, lv2_idx, long_click, long_cart, long_buy, long_buy_level2, long_cart_level2, long_click_level2, short_click, short_cart, short_buy, short_click_level2, short_cart_level2, short_buy_level2]
    tables = [T_context, T_back_cat, T_goods_rt, T_pair_click, T_passtime, T_user_group, T_goods, T_bucket_user, T_bucket_goods, T_bucket_goods_gross, T_pair, T_bucket_pair, T_bucket_user_cspu, T_bucket_ozid_cspu, T_bucket_user_behavior, T_cspu, T_supplier, T_level2]

    for t in BIG_TABLES:
        tables[t] = _to_rowmajor(tables[t])

    def src_part(t):
        cat = jnp.concatenate(
            [idxs[pos].astype(jnp.int32).reshape(B, n)
             for pos, n, _ in TBL_FEATS[t]], axis=1)
        return _pad_worker_chunks(cat)

    src_a = jnp.concatenate([src_part(t) for t in GROUP_A], axis=1)
    src_b = jnp.concatenate([src_part(t) for t in GROUP_B], axis=1)

    ka, kb = _kernels()
    out_a = ka(src_a, jnp.asarray(_DST["A"]), *[tables[t] for t in GROUP_A])
    out_ref = jax.new_ref(out_a)
    kb(src_b, jnp.asarray(_DST["B"]), *[tables[t] for t in GROUP_B], out_ref)
    return out_ref[...].reshape(B, NCOLS * D)


# speedup vs baseline: 1.0708x; 1.0708x over previous
import functools
import numpy as np
import jax
import jax.numpy as jnp
from jax import lax
from jax.experimental import pallas as pl
from jax.experimental.pallas import tpu as pltpu, tpu_sc as plsc

B = 1024
D = 32
NC, NS = 2, 16
NW = NC * NS
RPW = B // NW
CSZ = 128
NBUF = 3

FEATS = [
    (0, 5, 0), (1, 20, 1), (2, 20, 2), (3, 1, 3), (4, 20, 4), (5, 4, 5),
    (6, 26, 6), (7, 10, 7), (8, 10, 8), (9, 10, 9), (10, 10, 10),
    (11, 10, 11), (12, 10, 12), (13, 10, 13), (14, 10, 14),
    (15, 1, 15), (16, 1, 16), (17, 1, 17),
    (18, 50, 15), (19, 50, 15), (20, 50, 15),
    (21, 50, 17), (22, 50, 17), (23, 50, 17),
    (24, 20, 15), (25, 20, 15), (26, 20, 15),
    (27, 20, 17), (28, 20, 17), (29, 20, 17),
]
NCOLS = sum(n for _, n, _ in FEATS)

_off = 0
COL_OFF = []
for _, n, _ in FEATS:
    COL_OFF.append(_off)
    _off += n

TBL_FEATS = [[] for _ in range(18)]
for fi, (pos, n, t) in enumerate(FEATS):
    TBL_FEATS[t].append((pos, n, COL_OFF[fi]))
NT = [sum(n for _, n, _ in fs) for fs in TBL_FEATS]
CNT = [-(-RPW * nt // CSZ) for nt in NT]

GROUP_B = [15]
GROUP_A = [t for t in range(18) if t not in GROUP_B]

BIG_TABLES = [2, 6, 15, 16]

_BC = 512
_SELM = np.zeros((4, _BC // 4, _BC), np.float32)
for _a in range(4):
    _SELM[_a, np.arange(_BC // 4), np.arange(_BC // 4) * 4 + _a] = 1.0


@functools.cache
def _tconv_kernel(V):
    mb = 8
    step = mb * _BC
    nfull, rem = divmod(V, step)
    grid = nfull + (1 if rem else 0)

    def tbody(sel_ref, x_ref, tail_ref, o_ref):
        x8 = x_ref[...]
        if rem:
            x8 = jnp.where(pl.program_id(0) == grid - 1, tail_ref[...], x8)
        for k in range(mb):
            xk = x8[:, k * _BC:(k + 1) * _BC]
            parts = [jax.lax.dot_general(
                         sel_ref[a], xk, (((1,), (1,)), ((), ())),
                         preferred_element_type=jnp.float32)
                     for a in range(4)]
            o_ref[k * (_BC // 4):(k + 1) * (_BC // 4), :] = (
                jnp.concatenate(parts, axis=1))

    return pl.pallas_call(
        tbody,
        grid=(grid,),
        in_specs=[pl.BlockSpec((4, _BC // 4, _BC), lambda c: (0, 0, 0)),
                  pl.BlockSpec((32, step), lambda c: (0, c)),
                  pl.BlockSpec((32, step), lambda c: (0, 0))],
        out_specs=pl.BlockSpec((step // 4, 128), lambda c: (c, 0)),
        out_shape=jax.ShapeDtypeStruct((V * D // 128, 128), jnp.float32),
    )


def _to_rowmajor(T):
    V = T.shape[0]
    Tt = jnp.transpose(T)
    step = 8 * _BC
    rem = V % step
    if rem:
        tail = jnp.pad(Tt[:, V - rem:], ((0, 0), (0, step - rem)))
    else:
        tail = jnp.zeros((D, step), jnp.float32)
    q = _tconv_kernel(V)(jnp.asarray(_SELM), Tt, tail)
    return jnp.reshape(q, (V, D))


def _pad_worker_chunks(a):
    nt = a.shape[1]
    a = a.reshape(NW, RPW * nt)
    pad = -(-RPW * nt // CSZ) * CSZ - RPW * nt
    if pad:
        mod = jnp if isinstance(a, jax.Array) else np
        a = mod.pad(a, ((0, 0), (0, pad)), mode="edge")
    return a.reshape(NW, -1, CSZ)


def _dst_part(t):
    cols = np.concatenate([off + np.arange(n) for _, n, off in TBL_FEATS[t]])
    dst = (np.arange(B) * NCOLS)[:, None] + cols[None, :]
    return _pad_worker_chunks(dst.astype(np.int32))


_DST = {g: np.concatenate([_dst_part(t) for t in grp], axis=1)
        for g, grp in (("A", GROUP_A), ("B", GROUP_B))}


def _pipeline(tables, out_hbm, src_v, dst_v, bufs, gsems, ssems, tbl_of):
    n = len(tbl_of)

    def gather(i):
        s = i % NBUF
        return pltpu.async_copy(
            tables[tbl_of[i]].at[src_v.at[i]], bufs[s], gsems[s])

    def scatter(i):
        s = i % NBUF
        return pltpu.async_copy(bufs[s], out_hbm.at[dst_v.at[i]], ssems[s])

    g = [None] * n
    sc = [None] * n
    for i in range(min(NBUF - 1, n)):
        g[i] = gather(i)
    for i in range(n):
        if i >= 1:
            sc[i - 1].wait()
        if i + NBUF - 1 < n:
            g[i + NBUF - 1] = gather(i + NBUF - 1)
        g[i].wait()
        sc[i] = scatter(i)
    sc[n - 1].wait()


def _make_kernel(group, produce_out):
    grp = GROUP_A if group == "A" else GROUP_B
    ntab = len(grp)
    tbl_of = []
    for k, t in enumerate(grp):
        tbl_of += [k] * CNT[t]
    ch = len(tbl_of)
    mesh = plsc.VectorSubcoreMesh(core_axis_name="c", subcore_axis_name="s")

    @functools.partial(
        pl.kernel,
        out_type=(jax.ShapeDtypeStruct((B * NCOLS, D), jnp.float32)
                  if produce_out else ()),
        mesh=mesh,
        compiler_params=pltpu.CompilerParams(use_tc_tiling_on_sc=False),
        scratch_types=[
            pltpu.VMEM((ch, CSZ), jnp.int32),
            pltpu.VMEM((ch, CSZ), jnp.int32),
        ] + [pltpu.VMEM((CSZ, D), jnp.float32)] * NBUF
          + [pltpu.SemaphoreType.DMA] * (2 * NBUF),
    )
    def body(src_hbm, dst_hbm, *rest):
        tables = rest[:ntab]
        out_hbm = rest[ntab]
        src_v, dst_v = rest[ntab + 1:ntab + 3]
        bufs = rest[ntab + 3:ntab + 3 + NBUF]
        gsems = rest[ntab + 3 + NBUF:ntab + 3 + 2 * NBUF]
        ssems = rest[ntab + 3 + 2 * NBUF:ntab + 3 + 3 * NBUF]
        w = lax.axis_index("s") * NC + lax.axis_index("c")
        pltpu.sync_copy(src_hbm.at[w], src_v)
        pltpu.sync_copy(dst_hbm.at[w], dst_v)
        _pipeline(tables, out_hbm, src_v, dst_v, bufs, gsems, ssems, tbl_of)

    return body


@functools.cache
def _kernels():
    return _make_kernel("A", True), _make_kernel("B", False)


def kernel(context_features, realtime_back_category, realtime_goods, realtime_pair_click, realtime_passtime, realtime_user_group, goods_sparse, bucket_user_box_obj, bucket_goods_box_obj, bucket_goods_gross_obj, pair_feature, bucket_pair_box_obj, bucket_user_cspu_obj, bucket_ozid_cspu_obj, bucket_user_behavior_obj, cspu_idx, supplier_idx, lv2_idx, long_click, long_cart, long_buy, long_buy_level2, long_cart_level2, long_click_level2, short_click, short_cart, short_buy, short_click_level2, short_cart_level2, short_buy_level2, T_context, T_back_cat, T_goods_rt, T_pair_click, T_passtime, T_user_group, T_goods, T_bucket_user, T_bucket_goods, T_bucket_goods_gross, T_pair, T_bucket_pair, T_bucket_user_cspu, T_bucket_ozid_cspu, T_bucket_user_behavior, T_cspu, T_supplier, T_level2):
    idxs = [context_features, realtime_back_category, realtime_goods, realtime_pair_click, realtime_passtime, realtime_user_group, goods_sparse, bucket_user_box_obj, bucket_goods_box_obj, bucket_goods_gross_obj, pair_feature, bucket_pair_box_obj, bucket_user_cspu_obj, bucket_ozid_cspu_obj, bucket_user_behavior_obj, cspu_idx, supplier_idx, lv2_idx, long_click, long_cart, long_buy, long_buy_level2, long_cart_level2, long_click_level2, short_click, short_cart, short_buy, short_click_level2, short_cart_level2, short_buy_level2]
    tables = [T_context, T_back_cat, T_goods_rt, T_pair_click, T_passtime, T_user_group, T_goods, T_bucket_user, T_bucket_goods, T_bucket_goods_gross, T_pair, T_bucket_pair, T_bucket_user_cspu, T_bucket_ozid_cspu, T_bucket_user_behavior, T_cspu, T_supplier, T_level2]

    for t in BIG_TABLES:
        tables[t] = _to_rowmajor(tables[t])

    def src_part(t):
        cat = jnp.concatenate(
            [idxs[pos].astype(jnp.int32).reshape(B, n)
             for pos, n, _ in TBL_FEATS[t]], axis=1)
        return _pad_worker_chunks(cat)

    src_a = jnp.concatenate([src_part(t) for t in GROUP_A], axis=1)
    src_b = jnp.concatenate([src_part(t) for t in GROUP_B], axis=1)

    ka, kb = _kernels()
    out_a = ka(src_a, jnp.asarray(_DST["A"]), *[tables[t] for t in GROUP_A])
    out_ref = jax.new_ref(out_a)
    kb(src_b, jnp.asarray(_DST["B"]), *[tables[t] for t in GROUP_B], out_ref)
    return out_ref[...].reshape(B, NCOLS * D)
